# Initial kernel scaffold; baseline (speedup 1.0000x reference)
import functools

import jax
import jax.numpy as jnp
from jax import lax
from jax.experimental import pallas as pl
from jax.experimental.pallas import tpu as pltpu
from jax.experimental.pallas import tpu_sc as plsc

HID = 128
HALF = 64
N_NODES = 10000
N_HEDGES = 20000
NNZ = 320000
NC = 2
NS = 16
CS = 80

PAIRS_PER_TILE = NNZ // NS
NCHUNK = PAIRS_PER_TILE // CS
ROWS_E_TILE = N_HEDGES // NS
ROWS_V_TILE = N_NODES // NS

_MESH = plsc.VectorSubcoreMesh(core_axis_name="c", subcore_axis_name="s")


@functools.partial(
    pl.kernel,
    out_type=(
        jax.ShapeDtypeStruct((NC, N_HEDGES, HALF), jnp.float32),
        jax.ShapeDtypeStruct((N_HEDGES, 16), jnp.float32),
        jax.ShapeDtypeStruct((N_NODES, 16), jnp.float32),
    ),
    mesh=_MESH,
    scratch_types=(
        pltpu.VMEM((CS,), jnp.int32),
        pltpu.VMEM((CS,), jnp.int32),
        pltpu.VMEM((CS,), jnp.int32),
        pltpu.VMEM((CS, HALF), jnp.float32),
        pltpu.VMEM((CS, 16), jnp.float32),
        pltpu.VMEM_SHARED((N_HEDGES, HALF), jnp.float32),
        pltpu.VMEM_SHARED((N_HEDGES, 16), jnp.float32),
        pltpu.VMEM_SHARED((N_NODES, 16), jnp.float32),
        pltpu.SemaphoreType.DMA,
    ),
)
def _sc_pass1(xcat, vertex, edges, z64, z16, ones_h,
              s_out, ce_out, cv_out,
              vidx, eidx, gidx, rows, ones_v, acc_s, acc_ce, acc_cv, sem):
    c = lax.axis_index("c")
    s = lax.axis_index("s")
    pltpu.sync_copy(z64, acc_s.at[pl.ds(s * ROWS_E_TILE, ROWS_E_TILE)])
    pltpu.sync_copy(z16, acc_ce.at[pl.ds(s * ROWS_E_TILE, ROWS_E_TILE)])
    pltpu.sync_copy(z16.at[pl.ds(0, ROWS_V_TILE)],
                    acc_cv.at[pl.ds(s * ROWS_V_TILE, ROWS_V_TILE)])
    pltpu.sync_copy(ones_h, ones_v)
    plsc.subcore_barrier()

    voff = c * N_NODES
    base = s * PAIRS_PER_TILE

    def body(i, carry):
        off = base + i * CS
        pltpu.sync_copy(vertex.at[pl.ds(off, CS)], vidx)
        pltpu.sync_copy(edges.at[pl.ds(off, CS)], eidx)
        for j in range(CS // 16):
            sl = pl.ds(j * 16, 16)
            gidx[sl] = vidx[sl] + voff
        pltpu.async_copy(xcat.at[gidx], rows, sem).wait()
        pltpu.sync_copy(rows, acc_s.at[eidx], add=True)

        @pl.when(c == 0)
        def _():
            pltpu.sync_copy(ones_v, acc_ce.at[eidx], add=True)

        @pl.when(c == 1)
        def _():
            pltpu.sync_copy(ones_v, acc_cv.at[vidx], add=True)

        return carry

    lax.fori_loop(0, NCHUNK, body, 0)
    plsc.subcore_barrier()

    pltpu.sync_copy(acc_s.at[pl.ds(s * ROWS_E_TILE, ROWS_E_TILE)],
                    s_out.at[c].at[pl.ds(s * ROWS_E_TILE, ROWS_E_TILE)])

    @pl.when(c == 0)
    def _():
        pltpu.sync_copy(acc_ce.at[pl.ds(s * ROWS_E_TILE, ROWS_E_TILE)],
                        ce_out.at[pl.ds(s * ROWS_E_TILE, ROWS_E_TILE)])

    @pl.when(c == 1)
    def _():
        pltpu.sync_copy(acc_cv.at[pl.ds(s * ROWS_V_TILE, ROWS_V_TILE)],
                        cv_out.at[pl.ds(s * ROWS_V_TILE, ROWS_V_TILE)])


@functools.partial(
    pl.kernel,
    out_type=jax.ShapeDtypeStruct((NC, N_NODES, HALF), jnp.float32),
    mesh=_MESH,
    scratch_types=(
        pltpu.VMEM((CS,), jnp.int32),
        pltpu.VMEM((CS,), jnp.int32),
        pltpu.VMEM((CS,), jnp.int32),
        pltpu.VMEM((CS, HALF), jnp.float32),
        pltpu.VMEM_SHARED((N_NODES, HALF), jnp.float32),
        pltpu.SemaphoreType.DMA,
    ),
)
def _sc_pass2(ecat, vertex, edges, z64,
              v_out,
              vidx, eidx, gidx, rows, acc_v, sem):
    c = lax.axis_index("c")
    s = lax.axis_index("s")
    pltpu.sync_copy(z64.at[pl.ds(0, ROWS_V_TILE)],
                    acc_v.at[pl.ds(s * ROWS_V_TILE, ROWS_V_TILE)])
    plsc.subcore_barrier()

    eoff = c * N_HEDGES
    base = s * PAIRS_PER_TILE

    def body(i, carry):
        off = base + i * CS
        pltpu.sync_copy(vertex.at[pl.ds(off, CS)], vidx)
        pltpu.sync_copy(edges.at[pl.ds(off, CS)], eidx)
        for j in range(CS // 16):
            sl = pl.ds(j * 16, 16)
            gidx[sl] = eidx[sl] + eoff
        pltpu.async_copy(ecat.at[gidx], rows, sem).wait()
        pltpu.sync_copy(rows, acc_v.at[vidx], add=True)
        return carry

    lax.fori_loop(0, NCHUNK, body, 0)
    plsc.subcore_barrier()

    pltpu.sync_copy(acc_v.at[pl.ds(s * ROWS_V_TILE, ROWS_V_TILE)],
                    v_out.at[c].at[pl.ds(s * ROWS_V_TILE, ROWS_V_TILE)])


BR1 = 2000
BR2 = 2000


def _tc1_body(s_ref, cnt_ref, e_ref, w1a, w1b, b1, w2a, w2b, b2,
              enew_ref, esplit_ref):
    cnt = cnt_ref[:, 0:1]
    inv = 1.0 / jnp.maximum(cnt, 1.0)
    g = jnp.concatenate([s_ref[0], s_ref[1]], axis=1) * inv
    me = (jnp.dot(g, w1a[...], preferred_element_type=jnp.float32)
          + jnp.dot(e_ref[...], w1b[...], preferred_element_type=jnp.float32)
          + b1[...])
    me = jnp.where(cnt > 0.0, me, 0.0)
    en = (jnp.dot(e_ref[...], w2a[...], preferred_element_type=jnp.float32)
          + jnp.dot(me, w2b[...], preferred_element_type=jnp.float32)
          + b2[...])
    enew_ref[...] = en
    esplit_ref[0] = en[:, :HALF]
    esplit_ref[1] = en[:, HALF:]


_tc1 = pl.pallas_call(
    _tc1_body,
    grid=(N_HEDGES // BR1,),
    in_specs=[
        pl.BlockSpec((NC, BR1, HALF), lambda i: (0, i, 0)),
        pl.BlockSpec((BR1, 16), lambda i: (i, 0)),
        pl.BlockSpec((BR1, HID), lambda i: (i, 0)),
        pl.BlockSpec((HID, HID), lambda i: (0, 0)),
        pl.BlockSpec((HID, HID), lambda i: (0, 0)),
        pl.BlockSpec((1, HID), lambda i: (0, 0)),
        pl.BlockSpec((HID, HID), lambda i: (0, 0)),
        pl.BlockSpec((HID, HID), lambda i: (0, 0)),
        pl.BlockSpec((1, HID), lambda i: (0, 0)),
    ],
    out_specs=[
        pl.BlockSpec((BR1, HID), lambda i: (i, 0)),
        pl.BlockSpec((NC, BR1, HALF), lambda i: (0, i, 0)),
    ],
    out_shape=[
        jax.ShapeDtypeStruct((N_HEDGES, HID), jnp.float32),
        jax.ShapeDtypeStruct((NC, N_HEDGES, HALF), jnp.float32),
    ],
)


def _tc2_body(v_ref, cnt_ref, x_ref, w3a, w3b, b3, w4a, w4b, b4, xnew_ref):
    cnt = cnt_ref[:, 0:1]
    inv = 1.0 / jnp.maximum(cnt, 1.0)
    h = jnp.concatenate([v_ref[0], v_ref[1]], axis=1) * inv
    mv = (jnp.dot(x_ref[...], w3a[...], preferred_element_type=jnp.float32)
          + jnp.dot(h, w3b[...], preferred_element_type=jnp.float32)
          + b3[...])
    mv = jnp.where(cnt > 0.0, mv, 0.0)
    xnew_ref[...] = (jnp.dot(x_ref[...], w4a[...], preferred_element_type=jnp.float32)
                     + jnp.dot(mv, w4b[...], preferred_element_type=jnp.float32)
                     + b4[...])


_tc2 = pl.pallas_call(
    _tc2_body,
    grid=(N_NODES // BR2,),
    in_specs=[
        pl.BlockSpec((NC, BR2, HALF), lambda i: (0, i, 0)),
        pl.BlockSpec((BR2, 16), lambda i: (i, 0)),
        pl.BlockSpec((BR2, HID), lambda i: (i, 0)),
        pl.BlockSpec((HID, HID), lambda i: (0, 0)),
        pl.BlockSpec((HID, HID), lambda i: (0, 0)),
        pl.BlockSpec((1, HID), lambda i: (0, 0)),
        pl.BlockSpec((HID, HID), lambda i: (0, 0)),
        pl.BlockSpec((HID, HID), lambda i: (0, 0)),
        pl.BlockSpec((1, HID), lambda i: (0, 0)),
    ],
    out_specs=pl.BlockSpec((BR2, HID), lambda i: (i, 0)),
    out_shape=jax.ShapeDtypeStruct((N_NODES, HID), jnp.float32),
)


def kernel(X, E, vertex, edges, W1, b1, W2, b2, W3, b3, W4, b4):
    xcat = jnp.concatenate([X[:, :HALF], X[:, HALF:]], axis=0)
    z64 = jnp.zeros((ROWS_E_TILE, HALF), jnp.float32)
    z16 = jnp.zeros((ROWS_E_TILE, 16), jnp.float32)
    ones_h = jnp.ones((CS, 16), jnp.float32)

    s_acc, cnt_e, cnt_v = _sc_pass1(xcat, vertex, edges, z64, z16, ones_h)

    e_new, e_split = _tc1(
        s_acc, cnt_e, E,
        W1[:HID], W1[HID:], b1.reshape(1, HID),
        W2[:HID], W2[HID:], b2.reshape(1, HID),
    )

    ecat = e_split.reshape(NC * N_HEDGES, HALF)
    v_acc = _sc_pass2(ecat, vertex, edges, z64)

    x_new = _tc2(
        v_acc, cnt_v, X,
        W3[:HID], W3[HID:], b3.reshape(1, HID),
        W4[:HID], W4[HID:], b4.reshape(1, HID),
    )
    return x_new, e_new


# speedup vs baseline: 3.9709x; 3.9709x over previous
import functools

import jax
import jax.numpy as jnp
from jax import lax
from jax.experimental import pallas as pl
from jax.experimental.pallas import tpu as pltpu
from jax.experimental.pallas import tpu_sc as plsc

HID = 128
HALF = 64
N_NODES = 10000
N_HEDGES = 20000
NNZ = 320000
NC = 2
NS = 16
CS = 80

PAIRS_PER_TILE = NNZ // NS
NCHUNK = PAIRS_PER_TILE // CS
EPAD = 20480
VPAD = 10240
ROWS_E_TILE = EPAD // NS
ROWS_V_TILE = VPAD // NS

_MESH = plsc.VectorSubcoreMesh(core_axis_name="c", subcore_axis_name="s")


@functools.partial(
    pl.kernel,
    out_type=(
        jax.ShapeDtypeStruct((NC, EPAD, HALF), jnp.float32),
        jax.ShapeDtypeStruct((EPAD, 16), jnp.float32),
        jax.ShapeDtypeStruct((VPAD, 16), jnp.float32),
    ),
    mesh=_MESH,
    compiler_params=pltpu.CompilerParams(use_tc_tiling_on_sc=False),
    scratch_types=(
        pltpu.VMEM((CS,), jnp.int32),
        pltpu.VMEM((CS,), jnp.int32),
        pltpu.VMEM((CS,), jnp.int32),
        pltpu.VMEM((CS, HALF), jnp.float32),
        pltpu.VMEM((CS, 16), jnp.float32),
        pltpu.VMEM_SHARED((EPAD, HALF), jnp.float32),
        pltpu.VMEM_SHARED((EPAD, 16), jnp.float32),
        pltpu.VMEM_SHARED((VPAD, 16), jnp.float32),
        pltpu.SemaphoreType.DMA,
    ),
)
def _sc_pass1(xcat, vertex, edges, z64, z16, ones_h,
              s_out, ce_out, cv_out,
              vidx, eidx, gidx, rows, ones_v, acc_s, acc_ce, acc_cv, sem):
    c = lax.axis_index("c")
    s = lax.axis_index("s")
    pltpu.sync_copy(z64, acc_s.at[pl.ds(s * ROWS_E_TILE, ROWS_E_TILE)])
    pltpu.sync_copy(z16, acc_ce.at[pl.ds(s * ROWS_E_TILE, ROWS_E_TILE)])
    pltpu.sync_copy(z16.at[pl.ds(0, ROWS_V_TILE)],
                    acc_cv.at[pl.ds(s * ROWS_V_TILE, ROWS_V_TILE)])
    pltpu.sync_copy(ones_h, ones_v)
    plsc.subcore_barrier()

    voff = c * N_NODES
    base = s * PAIRS_PER_TILE

    def body(i, carry):
        off = base + i * CS
        pltpu.sync_copy(vertex.at[pl.ds(off, CS)], vidx)
        pltpu.sync_copy(edges.at[pl.ds(off, CS)], eidx)
        for j in range(CS // 16):
            sl = pl.ds(j * 16, 16)
            gidx[sl] = vidx[sl] + voff
        pltpu.async_copy(xcat.at[gidx], rows, sem).wait()
        pltpu.sync_copy(rows, acc_s.at[eidx], add=True)

        @pl.when(c == 0)
        def _():
            pltpu.sync_copy(ones_v, acc_ce.at[eidx], add=True)

        @pl.when(c == 1)
        def _():
            pltpu.sync_copy(ones_v, acc_cv.at[vidx], add=True)

        return carry

    lax.fori_loop(0, NCHUNK, body, 0)
    plsc.subcore_barrier()

    pltpu.sync_copy(acc_s.at[pl.ds(s * ROWS_E_TILE, ROWS_E_TILE)],
                    s_out.at[c].at[pl.ds(s * ROWS_E_TILE, ROWS_E_TILE)])

    @pl.when(c == 0)
    def _():
        pltpu.sync_copy(acc_ce.at[pl.ds(s * ROWS_E_TILE, ROWS_E_TILE)],
                        ce_out.at[pl.ds(s * ROWS_E_TILE, ROWS_E_TILE)])

    @pl.when(c == 1)
    def _():
        pltpu.sync_copy(acc_cv.at[pl.ds(s * ROWS_V_TILE, ROWS_V_TILE)],
                        cv_out.at[pl.ds(s * ROWS_V_TILE, ROWS_V_TILE)])


@functools.partial(
    pl.kernel,
    out_type=jax.ShapeDtypeStruct((NC, VPAD, HALF), jnp.float32),
    mesh=_MESH,
    compiler_params=pltpu.CompilerParams(use_tc_tiling_on_sc=False),
    scratch_types=(
        pltpu.VMEM((CS,), jnp.int32),
        pltpu.VMEM((CS,), jnp.int32),
        pltpu.VMEM((CS,), jnp.int32),
        pltpu.VMEM((CS, HALF), jnp.float32),
        pltpu.VMEM_SHARED((VPAD, HALF), jnp.float32),
        pltpu.SemaphoreType.DMA,
    ),
)
def _sc_pass2(ecat, vertex, edges, z64,
              v_out,
              vidx, eidx, gidx, rows, acc_v, sem):
    c = lax.axis_index("c")
    s = lax.axis_index("s")
    pltpu.sync_copy(z64.at[pl.ds(0, ROWS_V_TILE)],
                    acc_v.at[pl.ds(s * ROWS_V_TILE, ROWS_V_TILE)])
    plsc.subcore_barrier()

    eoff = c * N_HEDGES
    base = s * PAIRS_PER_TILE

    def body(i, carry):
        off = base + i * CS
        pltpu.sync_copy(vertex.at[pl.ds(off, CS)], vidx)
        pltpu.sync_copy(edges.at[pl.ds(off, CS)], eidx)
        for j in range(CS // 16):
            sl = pl.ds(j * 16, 16)
            gidx[sl] = eidx[sl] + eoff
        pltpu.async_copy(ecat.at[gidx], rows, sem).wait()
        pltpu.sync_copy(rows, acc_v.at[vidx], add=True)
        return carry

    lax.fori_loop(0, NCHUNK, body, 0)
    plsc.subcore_barrier()

    pltpu.sync_copy(acc_v.at[pl.ds(s * ROWS_V_TILE, ROWS_V_TILE)],
                    v_out.at[c].at[pl.ds(s * ROWS_V_TILE, ROWS_V_TILE)])


BR1 = 2000
BR2 = 2000


def _tc1_body(s_ref, cnt_ref, e_ref, w1a, w1b, b1, w2a, w2b, b2,
              enew_ref, esplit_ref):
    cnt = cnt_ref[:, 0:1]
    inv = 1.0 / jnp.maximum(cnt, 1.0)
    g = jnp.concatenate([s_ref[0], s_ref[1]], axis=1) * inv
    me = (jnp.dot(g, w1a[...], preferred_element_type=jnp.float32)
          + jnp.dot(e_ref[...], w1b[...], preferred_element_type=jnp.float32)
          + b1[...])
    me = jnp.where(cnt > 0.0, me, 0.0)
    en = (jnp.dot(e_ref[...], w2a[...], preferred_element_type=jnp.float32)
          + jnp.dot(me, w2b[...], preferred_element_type=jnp.float32)
          + b2[...])
    enew_ref[...] = en
    esplit_ref[0] = en[:, :HALF]
    esplit_ref[1] = en[:, HALF:]


_tc1 = pl.pallas_call(
    _tc1_body,
    grid=(N_HEDGES // BR1,),
    in_specs=[
        pl.BlockSpec((NC, BR1, HALF), lambda i: (0, i, 0)),
        pl.BlockSpec((BR1, 16), lambda i: (i, 0)),
        pl.BlockSpec((BR1, HID), lambda i: (i, 0)),
        pl.BlockSpec((HID, HID), lambda i: (0, 0)),
        pl.BlockSpec((HID, HID), lambda i: (0, 0)),
        pl.BlockSpec((1, HID), lambda i: (0, 0)),
        pl.BlockSpec((HID, HID), lambda i: (0, 0)),
        pl.BlockSpec((HID, HID), lambda i: (0, 0)),
        pl.BlockSpec((1, HID), lambda i: (0, 0)),
    ],
    out_specs=[
        pl.BlockSpec((BR1, HID), lambda i: (i, 0)),
        pl.BlockSpec((NC, BR1, HALF), lambda i: (0, i, 0)),
    ],
    out_shape=[
        jax.ShapeDtypeStruct((N_HEDGES, HID), jnp.float32),
        jax.ShapeDtypeStruct((NC, N_HEDGES, HALF), jnp.float32),
    ],
)


def _tc2_body(v_ref, cnt_ref, x_ref, w3a, w3b, b3, w4a, w4b, b4, xnew_ref):
    cnt = cnt_ref[:, 0:1]
    inv = 1.0 / jnp.maximum(cnt, 1.0)
    h = jnp.concatenate([v_ref[0], v_ref[1]], axis=1) * inv
    mv = (jnp.dot(x_ref[...], w3a[...], preferred_element_type=jnp.float32)
          + jnp.dot(h, w3b[...], preferred_element_type=jnp.float32)
          + b3[...])
    mv = jnp.where(cnt > 0.0, mv, 0.0)
    xnew_ref[...] = (jnp.dot(x_ref[...], w4a[...], preferred_element_type=jnp.float32)
                     + jnp.dot(mv, w4b[...], preferred_element_type=jnp.float32)
                     + b4[...])


_tc2 = pl.pallas_call(
    _tc2_body,
    grid=(N_NODES // BR2,),
    in_specs=[
        pl.BlockSpec((NC, BR2, HALF), lambda i: (0, i, 0)),
        pl.BlockSpec((BR2, 16), lambda i: (i, 0)),
        pl.BlockSpec((BR2, HID), lambda i: (i, 0)),
        pl.BlockSpec((HID, HID), lambda i: (0, 0)),
        pl.BlockSpec((HID, HID), lambda i: (0, 0)),
        pl.BlockSpec((1, HID), lambda i: (0, 0)),
        pl.BlockSpec((HID, HID), lambda i: (0, 0)),
        pl.BlockSpec((HID, HID), lambda i: (0, 0)),
        pl.BlockSpec((1, HID), lambda i: (0, 0)),
    ],
    out_specs=pl.BlockSpec((BR2, HID), lambda i: (i, 0)),
    out_shape=jax.ShapeDtypeStruct((N_NODES, HID), jnp.float32),
)


def kernel(X, E, vertex, edges, W1, b1, W2, b2, W3, b3, W4, b4):
    xcat = jnp.concatenate([X[:, :HALF], X[:, HALF:]], axis=0)
    z64 = jnp.zeros((ROWS_E_TILE, HALF), jnp.float32)
    z16 = jnp.zeros((ROWS_E_TILE, 16), jnp.float32)
    ones_h = jnp.ones((CS, 16), jnp.float32)

    s_acc, cnt_e, cnt_v = _sc_pass1(xcat, vertex, edges, z64, z16, ones_h)

    e_new, e_split = _tc1(
        s_acc, cnt_e, E,
        W1[:HID], W1[HID:], b1.reshape(1, HID),
        W2[:HID], W2[HID:], b2.reshape(1, HID),
    )

    ecat = e_split.reshape(NC * N_HEDGES, HALF)
    v_acc = _sc_pass2(ecat, vertex, edges, z64)

    x_new = _tc2(
        v_acc, cnt_v, X,
        W3[:HID], W3[HID:], b3.reshape(1, HID),
        W4[:HID], W4[HID:], b4.reshape(1, HID),
    )
    return x_new, e_new


# speedup vs baseline: 5.3634x; 1.3507x over previous
import functools

import jax
import jax.numpy as jnp
from jax import lax
from jax.experimental import pallas as pl
from jax.experimental.pallas import tpu as pltpu
from jax.experimental.pallas import tpu_sc as plsc

HID = 128
HALF = 64
N_NODES = 10000
N_HEDGES = 20000
NNZ = 320000
NC = 2
NS = 16
CS = 80
K = 5

PAIRS_PER_TILE = NNZ // NS
NCHUNK = PAIRS_PER_TILE // CS
NGRP2 = NCHUNK // (2 * K)
ADJ_STEPS = PAIRS_PER_TILE // 16
EPAD = 20480
VPAD = 10240
ROWS_E_TILE = EPAD // NS
ROWS_V_TILE = VPAD // NS

_MESH = plsc.VectorSubcoreMesh(core_axis_name="c", subcore_axis_name="s")
_SC_PARAMS = pltpu.CompilerParams(use_tc_tiling_on_sc=False)


def _stage_idx(idx3_list, slot, g, dsts):
    for src3, dst in zip(idx3_list, dsts):
        pltpu.sync_copy(src3, dst.at[slot])


def _adjust_slot(gidx2, slot, off):
    for j in range(CS // 16):
        sl = pl.ds(j * 16, 16)
        gidx2[slot, sl] = gidx2[slot, sl] + off


@functools.partial(
    pl.kernel,
    out_type=(
        jax.ShapeDtypeStruct((NC, EPAD, HALF), jnp.float32),
        jax.ShapeDtypeStruct((EPAD, 16), jnp.float32),
        jax.ShapeDtypeStruct((VPAD, 16), jnp.float32),
    ),
    mesh=_MESH,
    compiler_params=_SC_PARAMS,
    scratch_types=(
        pltpu.VMEM((2, CS), jnp.int32),
        pltpu.VMEM((2, CS), jnp.int32),
        pltpu.VMEM((2, CS), jnp.int32),
        pltpu.VMEM((2, CS, HALF), jnp.float32),
        pltpu.VMEM((CS, 16), jnp.float32),
        pltpu.VMEM_SHARED((EPAD, HALF), jnp.float32),
        pltpu.VMEM_SHARED((EPAD, 16), jnp.float32),
        pltpu.VMEM_SHARED((VPAD, 16), jnp.float32),
        pltpu.SemaphoreType.DMA,
        pltpu.SemaphoreType.DMA,
        pltpu.SemaphoreType.DMA,
    ),
)
def _sc_pass1(xcat, vertex3, edges3, z64, z16, ones_h,
              s_out, ce_out, cv_out,
              vidx2, eidx2, gidx2, rows, ones_v,
              acc_s, acc_ce, acc_cv, gsem, ssem, csem):
    c = lax.axis_index("c")
    s = lax.axis_index("s")
    pltpu.sync_copy(ones_h, ones_v)
    pltpu.sync_copy(z64, acc_s.at[pl.ds(s * ROWS_E_TILE, ROWS_E_TILE)])
    pltpu.sync_copy(z16, acc_ce.at[pl.ds(s * ROWS_E_TILE, ROWS_E_TILE)])
    pltpu.sync_copy(z16.at[pl.ds(0, ROWS_V_TILE)],
                    acc_cv.at[pl.ds(s * ROWS_V_TILE, ROWS_V_TILE)])
    pltpu.sync_copy(vertex3.at[s, 0], vidx2.at[0])
    pltpu.sync_copy(edges3.at[s, 0], eidx2.at[0])
    pltpu.sync_copy(vertex3.at[s, 0], gidx2.at[0])
    _adjust_slot(gidx2, 0, c * N_NODES)
    plsc.subcore_barrier()
    pltpu.async_copy(xcat.at[gidx2.at[0]], rows.at[0], gsem)

    def pair(it, carry):
        for p in (0, 1):
            g = it * 2 + p
            @pl.when(g >= 1)
            def _():
                pltpu.make_async_copy(xcat.at[pl.ds(0, CS)], rows.at[1 - p], ssem).wait()
                pltpu.make_async_copy(z16.at[pl.ds(0, CS)], ones_v, csem).wait()

            @pl.when(g + 1 < NCHUNK)
            def _():
                g1 = g + 1
                pltpu.sync_copy(vertex3.at[s, g1], vidx2.at[1 - p])
                pltpu.sync_copy(edges3.at[s, g1], eidx2.at[1 - p])
                pltpu.sync_copy(vertex3.at[s, g1], gidx2.at[1 - p])
            _adjust_slot(gidx2, 1 - p, c * N_NODES)

            pltpu.make_async_copy(xcat.at[pl.ds(0, CS)], rows.at[p], gsem).wait()

            @pl.when(g + 1 < NCHUNK)
            def _():
                pltpu.async_copy(xcat.at[gidx2.at[1 - p]], rows.at[1 - p], gsem)

            pltpu.async_copy(rows.at[p], acc_s.at[eidx2.at[p]], ssem, add=True)

            @pl.when(c == 0)
            def _():
                pltpu.async_copy(ones_v, acc_ce.at[eidx2.at[p]], csem, add=True)

            @pl.when(c == 1)
            def _():
                pltpu.async_copy(ones_v, acc_cv.at[vidx2.at[p]], csem, add=True)
        return carry

    lax.fori_loop(0, NCHUNK // 2, pair, 0)
    pltpu.make_async_copy(xcat.at[pl.ds(0, CS)], rows.at[1], ssem).wait()
    pltpu.make_async_copy(z16.at[pl.ds(0, CS)], ones_v, csem).wait()
    plsc.subcore_barrier()

    pltpu.sync_copy(acc_s.at[pl.ds(s * ROWS_E_TILE, ROWS_E_TILE)],
                    s_out.at[c].at[pl.ds(s * ROWS_E_TILE, ROWS_E_TILE)])

    @pl.when(c == 0)
    def _():
        pltpu.sync_copy(acc_ce.at[pl.ds(s * ROWS_E_TILE, ROWS_E_TILE)],
                        ce_out.at[pl.ds(s * ROWS_E_TILE, ROWS_E_TILE)])

    @pl.when(c == 1)
    def _():
        pltpu.sync_copy(acc_cv.at[pl.ds(s * ROWS_V_TILE, ROWS_V_TILE)],
                        cv_out.at[pl.ds(s * ROWS_V_TILE, ROWS_V_TILE)])


@functools.partial(
    pl.kernel,
    out_type=jax.ShapeDtypeStruct((NC, VPAD, HALF), jnp.float32),
    mesh=_MESH,
    compiler_params=_SC_PARAMS,
    scratch_types=(
        pltpu.VMEM((2, CS), jnp.int32),
        pltpu.VMEM((2, CS), jnp.int32),
        pltpu.VMEM((2, CS, HALF), jnp.float32),
        pltpu.VMEM_SHARED((VPAD, HALF), jnp.float32),
        pltpu.SemaphoreType.DMA,
        pltpu.SemaphoreType.DMA,
    ),
)
def _sc_pass2(ecat, vertex3, edges3, z64,
              v_out,
              vidx2, gidx2, rows, acc_v, gsem, ssem):
    c = lax.axis_index("c")
    s = lax.axis_index("s")
    pltpu.sync_copy(z64.at[pl.ds(0, ROWS_V_TILE)],
                    acc_v.at[pl.ds(s * ROWS_V_TILE, ROWS_V_TILE)])
    pltpu.sync_copy(vertex3.at[s, 0], vidx2.at[0])
    pltpu.sync_copy(edges3.at[s, 0], gidx2.at[0])
    _adjust_slot(gidx2, 0, c * N_HEDGES)
    plsc.subcore_barrier()
    pltpu.async_copy(ecat.at[gidx2.at[0]], rows.at[0], gsem)

    def pair(it, carry):
        for p in (0, 1):
            g = it * 2 + p

            @pl.when(g >= 1)
            def _():
                pltpu.make_async_copy(ecat.at[pl.ds(0, CS)], rows.at[1 - p], ssem).wait()

            @pl.when(g + 1 < NCHUNK)
            def _():
                g1 = g + 1
                pltpu.sync_copy(vertex3.at[s, g1], vidx2.at[1 - p])
                pltpu.sync_copy(edges3.at[s, g1], gidx2.at[1 - p])
            _adjust_slot(gidx2, 1 - p, c * N_HEDGES)

            pltpu.make_async_copy(ecat.at[pl.ds(0, CS)], rows.at[p], gsem).wait()

            @pl.when(g + 1 < NCHUNK)
            def _():
                pltpu.async_copy(ecat.at[gidx2.at[1 - p]], rows.at[1 - p], gsem)

            pltpu.async_copy(rows.at[p], acc_v.at[vidx2.at[p]], ssem, add=True)
        return carry

    lax.fori_loop(0, NCHUNK // 2, pair, 0)
    pltpu.make_async_copy(ecat.at[pl.ds(0, CS)], rows.at[1], ssem).wait()
    plsc.subcore_barrier()

    pltpu.sync_copy(acc_v.at[pl.ds(s * ROWS_V_TILE, ROWS_V_TILE)],
                    v_out.at[c].at[pl.ds(s * ROWS_V_TILE, ROWS_V_TILE)])


BR1 = 2000
BR2 = 2000


def _tc1_body(s_ref, cnt_ref, e_ref, w1a, w1b, b1, w2a, w2b, b2,
              enew_ref, esplit_ref):
    cnt = cnt_ref[:, 0:1]
    inv = 1.0 / jnp.maximum(cnt, 1.0)
    g = jnp.concatenate([s_ref[0], s_ref[1]], axis=1) * inv
    me = (jnp.dot(g, w1a[...], preferred_element_type=jnp.float32)
          + jnp.dot(e_ref[...], w1b[...], preferred_element_type=jnp.float32)
          + b1[...])
    me = jnp.where(cnt > 0.0, me, 0.0)
    en = (jnp.dot(e_ref[...], w2a[...], preferred_element_type=jnp.float32)
          + jnp.dot(me, w2b[...], preferred_element_type=jnp.float32)
          + b2[...])
    enew_ref[...] = en
    esplit_ref[0] = en[:, :HALF]
    esplit_ref[1] = en[:, HALF:]


_tc1 = pl.pallas_call(
    _tc1_body,
    grid=(N_HEDGES // BR1,),
    in_specs=[
        pl.BlockSpec((NC, BR1, HALF), lambda i: (0, i, 0)),
        pl.BlockSpec((BR1, 16), lambda i: (i, 0)),
        pl.BlockSpec((BR1, HID), lambda i: (i, 0)),
        pl.BlockSpec((HID, HID), lambda i: (0, 0)),
        pl.BlockSpec((HID, HID), lambda i: (0, 0)),
        pl.BlockSpec((1, HID), lambda i: (0, 0)),
        pl.BlockSpec((HID, HID), lambda i: (0, 0)),
        pl.BlockSpec((HID, HID), lambda i: (0, 0)),
        pl.BlockSpec((1, HID), lambda i: (0, 0)),
    ],
    out_specs=[
        pl.BlockSpec((BR1, HID), lambda i: (i, 0)),
        pl.BlockSpec((NC, BR1, HALF), lambda i: (0, i, 0)),
    ],
    out_shape=[
        jax.ShapeDtypeStruct((N_HEDGES, HID), jnp.float32),
        jax.ShapeDtypeStruct((NC, N_HEDGES, HALF), jnp.float32),
    ],
)


def _tc2_body(v_ref, cnt_ref, x_ref, w3a, w3b, b3, w4a, w4b, b4, xnew_ref):
    cnt = cnt_ref[:, 0:1]
    inv = 1.0 / jnp.maximum(cnt, 1.0)
    h = jnp.concatenate([v_ref[0], v_ref[1]], axis=1) * inv
    mv = (jnp.dot(x_ref[...], w3a[...], preferred_element_type=jnp.float32)
          + jnp.dot(h, w3b[...], preferred_element_type=jnp.float32)
          + b3[...])
    mv = jnp.where(cnt > 0.0, mv, 0.0)
    xnew_ref[...] = (jnp.dot(x_ref[...], w4a[...], preferred_element_type=jnp.float32)
                     + jnp.dot(mv, w4b[...], preferred_element_type=jnp.float32)
                     + b4[...])


_tc2 = pl.pallas_call(
    _tc2_body,
    grid=(N_NODES // BR2,),
    in_specs=[
        pl.BlockSpec((NC, BR2, HALF), lambda i: (0, i, 0)),
        pl.BlockSpec((BR2, 16), lambda i: (i, 0)),
        pl.BlockSpec((BR2, HID), lambda i: (i, 0)),
        pl.BlockSpec((HID, HID), lambda i: (0, 0)),
        pl.BlockSpec((HID, HID), lambda i: (0, 0)),
        pl.BlockSpec((1, HID), lambda i: (0, 0)),
        pl.BlockSpec((HID, HID), lambda i: (0, 0)),
        pl.BlockSpec((HID, HID), lambda i: (0, 0)),
        pl.BlockSpec((1, HID), lambda i: (0, 0)),
    ],
    out_specs=pl.BlockSpec((BR2, HID), lambda i: (i, 0)),
    out_shape=jax.ShapeDtypeStruct((N_NODES, HID), jnp.float32),
)


def kernel(X, E, vertex, edges, W1, b1, W2, b2, W3, b3, W4, b4):
    xcat = jnp.concatenate([X[:, :HALF], X[:, HALF:]], axis=0)
    vertex3 = vertex.reshape(NS, NCHUNK, CS)
    edges3 = edges.reshape(NS, NCHUNK, CS)
    z64 = jnp.zeros((ROWS_E_TILE, HALF), jnp.float32)
    z16 = jnp.zeros((ROWS_E_TILE, 16), jnp.float32)
    ones_h = jnp.ones((CS, 16), jnp.float32)

    s_acc, cnt_e, cnt_v = _sc_pass1(xcat, vertex3, edges3, z64, z16, ones_h)

    e_new, e_split = _tc1(
        s_acc, cnt_e, E,
        W1[:HID], W1[HID:], b1.reshape(1, HID),
        W2[:HID], W2[HID:], b2.reshape(1, HID),
    )

    ecat = e_split.reshape(NC * N_HEDGES, HALF)
    v_acc = _sc_pass2(ecat, vertex3, edges3, z64)

    x_new = _tc2(
        v_acc, cnt_v, X,
        W3[:HID], W3[HID:], b3.reshape(1, HID),
        W4[:HID], W4[HID:], b4.reshape(1, HID),
    )
    return x_new, e_new


# speedup vs baseline: 7.2205x; 1.3463x over previous
import functools

import jax
import jax.numpy as jnp
from jax import lax
from jax.experimental import pallas as pl
from jax.experimental.pallas import tpu as pltpu
from jax.experimental.pallas import tpu_sc as plsc

HID = 128
HALF = 64
N_NODES = 10000
N_HEDGES = 20000
NNZ = 320000
NC = 2
NS = 16

CREAL = 125
CS = 128
NCHUNK = 160
NB = 10
NBLK = NCHUNK // NB
PAIRS_PER_TILE = NNZ // NS
EPAD = 20480
VPAD = 10240
ROWS_E_TILE = EPAD // NS
ROWS_V_TILE = VPAD // NS
E_TRASH = 20400
VC_PAD = 10300
V_TRASH = 10200

_MESH = plsc.VectorSubcoreMesh(core_axis_name="c", subcore_axis_name="s")
_SC_PARAMS = pltpu.CompilerParams(use_tc_tiling_on_sc=False)


@functools.partial(
    pl.kernel,
    out_type=(
        jax.ShapeDtypeStruct((NC, EPAD, HALF), jnp.float32),
        jax.ShapeDtypeStruct((EPAD, 16), jnp.float32),
        jax.ShapeDtypeStruct((VPAD, 16), jnp.float32),
    ),
    mesh=_MESH,
    compiler_params=_SC_PARAMS,
    scratch_types=(
        pltpu.VMEM((2, NB, CS), jnp.int32),
        pltpu.VMEM((2, NB, CS), jnp.int32),
        pltpu.VMEM((2, NB, CS), jnp.int32),
        pltpu.VMEM((2, CS, HALF), jnp.float32),
        pltpu.VMEM((CS, 16), jnp.float32),
        pltpu.VMEM_SHARED((EPAD, HALF), jnp.float32),
        pltpu.VMEM_SHARED((EPAD, 16), jnp.float32),
        pltpu.SemaphoreType.DMA,
        pltpu.SemaphoreType.DMA,
        pltpu.SemaphoreType.DMA,
        pltpu.SemaphoreType.DMA,
    ),
)
def _sc_pass1(xcat, vg4, vga4, vc4, et4, z64, z16, ones_h,
              s_out, ce_out, cv_out,
              gblk, eblk, cblk, rows, ones_v,
              acc_s, cnt, gsem, ssem, csem, isem):
    c = lax.axis_index("c")
    s = lax.axis_index("s")
    pltpu.sync_copy(ones_h, ones_v)
    pltpu.sync_copy(z64, acc_s.at[pl.ds(s * ROWS_E_TILE, ROWS_E_TILE)])
    pltpu.sync_copy(z16, cnt.at[pl.ds(s * ROWS_E_TILE, ROWS_E_TILE)])
    pltpu.sync_copy(et4.at[s, 0], eblk.at[0])
    pltpu.sync_copy(vc4.at[s, 0], cblk.at[0])

    @pl.when(c == 0)
    def _():
        pltpu.sync_copy(vg4.at[s, 0], gblk.at[0])

    @pl.when(c == 1)
    def _():
        pltpu.sync_copy(vga4.at[s, 0], gblk.at[0])

    plsc.subcore_barrier()
    pltpu.async_copy(xcat.at[gblk.at[0, 0]], rows.at[0], gsem)

    def pair(it, carry):
        for p in (0, 1):
            g = it * 2 + p
            blk = g // NB
            slot = lax.rem(blk, 2)
            row = lax.rem(g, NB)

            @pl.when(g >= 1)
            def _():
                pltpu.make_async_copy(xcat.at[pl.ds(0, CS)], rows.at[1 - p], ssem).wait()
                pltpu.make_async_copy(z16.at[pl.ds(0, CS)], ones_v, csem).wait()

            @pl.when((lax.rem(g, NB) == 0) & (blk + 1 < NBLK))
            def _():
                nslot = lax.rem(blk + 1, 2)
                pltpu.async_copy(et4.at[s, blk + 1], eblk.at[nslot], isem)
                pltpu.async_copy(vc4.at[s, blk + 1], cblk.at[nslot], isem)

                @pl.when(c == 0)
                def _():
                    pltpu.async_copy(vg4.at[s, blk + 1], gblk.at[nslot], isem)

                @pl.when(c == 1)
                def _():
                    pltpu.async_copy(vga4.at[s, blk + 1], gblk.at[nslot], isem)

            pltpu.make_async_copy(xcat.at[pl.ds(0, CS)], rows.at[p], gsem).wait()

            g1 = g + 1

            @pl.when(g1 < NCHUNK)
            def _():
                s1 = lax.rem(g1 // NB, 2)
                r1 = lax.rem(g1, NB)

                @pl.when(lax.rem(g1, NB) == 0)
                def _():
                    for _u in range(3):
                        pltpu.make_async_copy(et4.at[s, 0], eblk.at[0], isem).wait()

                pltpu.async_copy(xcat.at[gblk.at[s1, r1]], rows.at[1 - p], gsem)

            pltpu.async_copy(rows.at[p], acc_s.at[eblk.at[slot, row]], ssem, add=True)

            @pl.when(c == 0)
            def _():
                pltpu.async_copy(ones_v, cnt.at[eblk.at[slot, row]], csem, add=True)

            @pl.when(c == 1)
            def _():
                pltpu.async_copy(ones_v, cnt.at[cblk.at[slot, row]], csem, add=True)

        return carry

    lax.fori_loop(0, NCHUNK // 2, pair, 0)
    pltpu.make_async_copy(xcat.at[pl.ds(0, CS)], rows.at[1], ssem).wait()
    pltpu.make_async_copy(z16.at[pl.ds(0, CS)], ones_v, csem).wait()
    plsc.subcore_barrier()

    pltpu.sync_copy(acc_s.at[pl.ds(s * ROWS_E_TILE, ROWS_E_TILE)],
                    s_out.at[c].at[pl.ds(s * ROWS_E_TILE, ROWS_E_TILE)])

    @pl.when(c == 0)
    def _():
        pltpu.sync_copy(cnt.at[pl.ds(s * ROWS_E_TILE, ROWS_E_TILE)],
                        ce_out.at[pl.ds(s * ROWS_E_TILE, ROWS_E_TILE)])

    @pl.when(c == 1)
    def _():
        pltpu.sync_copy(cnt.at[pl.ds(N_NODES + s * ROWS_V_TILE, ROWS_V_TILE)],
                        cv_out.at[pl.ds(s * ROWS_V_TILE, ROWS_V_TILE)])


@functools.partial(
    pl.kernel,
    out_type=jax.ShapeDtypeStruct((NC, VPAD, HALF), jnp.float32),
    mesh=_MESH,
    compiler_params=_SC_PARAMS,
    scratch_types=(
        pltpu.VMEM((2, NB, CS), jnp.int32),
        pltpu.VMEM((2, NB, CS), jnp.int32),
        pltpu.VMEM((2, CS, HALF), jnp.float32),
        pltpu.VMEM_SHARED((VPAD, HALF), jnp.float32),
        pltpu.SemaphoreType.DMA,
        pltpu.SemaphoreType.DMA,
        pltpu.SemaphoreType.DMA,
    ),
)
def _sc_pass2(ecat, ep4, epa4, vt4, z64,
              v_out,
              gblk, vblk, rows, acc_v, gsem, ssem, isem):
    c = lax.axis_index("c")
    s = lax.axis_index("s")
    pltpu.sync_copy(z64.at[pl.ds(0, ROWS_V_TILE)],
                    acc_v.at[pl.ds(s * ROWS_V_TILE, ROWS_V_TILE)])
    pltpu.sync_copy(vt4.at[s, 0], vblk.at[0])

    @pl.when(c == 0)
    def _():
        pltpu.sync_copy(ep4.at[s, 0], gblk.at[0])

    @pl.when(c == 1)
    def _():
        pltpu.sync_copy(epa4.at[s, 0], gblk.at[0])

    plsc.subcore_barrier()
    pltpu.async_copy(ecat.at[gblk.at[0, 0]], rows.at[0], gsem)

    def pair(it, carry):
        for p in (0, 1):
            g = it * 2 + p
            blk = g // NB
            slot = lax.rem(blk, 2)
            row = lax.rem(g, NB)

            @pl.when(g >= 1)
            def _():
                pltpu.make_async_copy(ecat.at[pl.ds(0, CS)], rows.at[1 - p], ssem).wait()

            @pl.when((lax.rem(g, NB) == 0) & (blk + 1 < NBLK))
            def _():
                nslot = lax.rem(blk + 1, 2)
                pltpu.async_copy(vt4.at[s, blk + 1], vblk.at[nslot], isem)

                @pl.when(c == 0)
                def _():
                    pltpu.async_copy(ep4.at[s, blk + 1], gblk.at[nslot], isem)

                @pl.when(c == 1)
                def _():
                    pltpu.async_copy(epa4.at[s, blk + 1], gblk.at[nslot], isem)

            pltpu.make_async_copy(ecat.at[pl.ds(0, CS)], rows.at[p], gsem).wait()

            g1 = g + 1

            @pl.when(g1 < NCHUNK)
            def _():
                s1 = lax.rem(g1 // NB, 2)
                r1 = lax.rem(g1, NB)

                @pl.when(lax.rem(g1, NB) == 0)
                def _():
                    for _u in range(2):
                        pltpu.make_async_copy(vt4.at[s, 0], vblk.at[0], isem).wait()

                pltpu.async_copy(ecat.at[gblk.at[s1, r1]], rows.at[1 - p], gsem)

            pltpu.async_copy(rows.at[p], acc_v.at[vblk.at[slot, row]], ssem, add=True)
        return carry

    lax.fori_loop(0, NCHUNK // 2, pair, 0)
    pltpu.make_async_copy(ecat.at[pl.ds(0, CS)], rows.at[1], ssem).wait()
    plsc.subcore_barrier()

    pltpu.sync_copy(acc_v.at[pl.ds(s * ROWS_V_TILE, ROWS_V_TILE)],
                    v_out.at[c].at[pl.ds(s * ROWS_V_TILE, ROWS_V_TILE)])


BR1 = 2000
BR2 = 2000


def _tc1_body(s_ref, cnt_ref, e_ref, w1a, w1b, b1, w2a, w2b, b2,
              enew_ref, esplit_ref):
    cnt = cnt_ref[:, 0:1]
    inv = 1.0 / jnp.maximum(cnt, 1.0)
    g = jnp.concatenate([s_ref[0], s_ref[1]], axis=1) * inv
    me = (jnp.dot(g, w1a[...], preferred_element_type=jnp.float32)
          + jnp.dot(e_ref[...], w1b[...], preferred_element_type=jnp.float32)
          + b1[...])
    me = jnp.where(cnt > 0.0, me, 0.0)
    en = (jnp.dot(e_ref[...], w2a[...], preferred_element_type=jnp.float32)
          + jnp.dot(me, w2b[...], preferred_element_type=jnp.float32)
          + b2[...])
    enew_ref[...] = en
    esplit_ref[0] = en[:, :HALF]
    esplit_ref[1] = en[:, HALF:]


_tc1 = pl.pallas_call(
    _tc1_body,
    grid=(N_HEDGES // BR1,),
    in_specs=[
        pl.BlockSpec((NC, BR1, HALF), lambda i: (0, i, 0)),
        pl.BlockSpec((BR1, 16), lambda i: (i, 0)),
        pl.BlockSpec((BR1, HID), lambda i: (i, 0)),
        pl.BlockSpec((HID, HID), lambda i: (0, 0)),
        pl.BlockSpec((HID, HID), lambda i: (0, 0)),
        pl.BlockSpec((1, HID), lambda i: (0, 0)),
        pl.BlockSpec((HID, HID), lambda i: (0, 0)),
        pl.BlockSpec((HID, HID), lambda i: (0, 0)),
        pl.BlockSpec((1, HID), lambda i: (0, 0)),
    ],
    out_specs=[
        pl.BlockSpec((BR1, HID), lambda i: (i, 0)),
        pl.BlockSpec((NC, BR1, HALF), lambda i: (0, i, 0)),
    ],
    out_shape=[
        jax.ShapeDtypeStruct((N_HEDGES, HID), jnp.float32),
        jax.ShapeDtypeStruct((NC, N_HEDGES, HALF), jnp.float32),
    ],
)


def _tc2_body(v_ref, cnt_ref, x_ref, w3a, w3b, b3, w4a, w4b, b4, xnew_ref):
    cnt = cnt_ref[:, 0:1]
    inv = 1.0 / jnp.maximum(cnt, 1.0)
    h = jnp.concatenate([v_ref[0], v_ref[1]], axis=1) * inv
    mv = (jnp.dot(x_ref[...], w3a[...], preferred_element_type=jnp.float32)
          + jnp.dot(h, w3b[...], preferred_element_type=jnp.float32)
          + b3[...])
    mv = jnp.where(cnt > 0.0, mv, 0.0)
    xnew_ref[...] = (jnp.dot(x_ref[...], w4a[...], preferred_element_type=jnp.float32)
                     + jnp.dot(mv, w4b[...], preferred_element_type=jnp.float32)
                     + b4[...])


_tc2 = pl.pallas_call(
    _tc2_body,
    grid=(N_NODES // BR2,),
    in_specs=[
        pl.BlockSpec((NC, BR2, HALF), lambda i: (0, i, 0)),
        pl.BlockSpec((BR2, 16), lambda i: (i, 0)),
        pl.BlockSpec((BR2, HID), lambda i: (i, 0)),
        pl.BlockSpec((HID, HID), lambda i: (0, 0)),
        pl.BlockSpec((HID, HID), lambda i: (0, 0)),
        pl.BlockSpec((1, HID), lambda i: (0, 0)),
        pl.BlockSpec((HID, HID), lambda i: (0, 0)),
        pl.BlockSpec((HID, HID), lambda i: (0, 0)),
        pl.BlockSpec((1, HID), lambda i: (0, 0)),
    ],
    out_specs=pl.BlockSpec((BR2, HID), lambda i: (i, 0)),
    out_shape=jax.ShapeDtypeStruct((N_NODES, HID), jnp.float32),
)


def kernel(X, E, vertex, edges, W1, b1, W2, b2, W3, b3, W4, b4):
    xcat = jnp.concatenate([X[:, :HALF], X[:, HALF:]], axis=0)
    v3 = vertex.reshape(NS, NCHUNK, CREAL)
    e3 = edges.reshape(NS, NCHUNK, CREAL)
    padw = ((0, 0), (0, 0), (0, CS - CREAL))
    shp4 = (NS, NBLK, NB, CS)
    vg4 = jnp.pad(v3, padw).reshape(shp4)
    vga4 = vg4 + N_NODES
    vc4 = (jnp.pad(v3, padw, constant_values=VC_PAD) + N_NODES).reshape(shp4)
    et4 = jnp.pad(e3, padw, constant_values=E_TRASH).reshape(shp4)
    ep4 = jnp.pad(e3, padw).reshape(shp4)
    epa4 = ep4 + N_HEDGES
    vt4 = jnp.pad(v3, padw, constant_values=V_TRASH).reshape(shp4)
    z64 = jnp.zeros((ROWS_E_TILE, HALF), jnp.float32)
    z16 = jnp.zeros((ROWS_E_TILE, 16), jnp.float32)
    ones_h = jnp.ones((CS, 16), jnp.float32)

    s_acc, cnt_e, cnt_v = _sc_pass1(xcat, vg4, vga4, vc4, et4,
                                    z64, z16, ones_h)

    e_new, e_split = _tc1(
        s_acc, cnt_e, E,
        W1[:HID], W1[HID:], b1.reshape(1, HID),
        W2[:HID], W2[HID:], b2.reshape(1, HID),
    )

    ecat = e_split.reshape(NC * N_HEDGES, HALF)
    v_acc = _sc_pass2(ecat, ep4, epa4, vt4, z64)

    x_new = _tc2(
        v_acc, cnt_v, X,
        W3[:HID], W3[HID:], b3.reshape(1, HID),
        W4[:HID], W4[HID:], b4.reshape(1, HID),
    )
    return x_new, e_new
